# baseline (speedup 1.0000x reference)
import functools

import jax
import jax.numpy as jnp
from jax import lax
from jax.experimental import pallas as pl
from jax.experimental.pallas import tpu as pltpu
from jax.experimental.pallas import tpu_sc as plsc

_N_BINS = 15
_LANES = 128
_BLOCK = 16384


def _dense_body(logits2_ref, labe_ref, labo_ref, packed_ref, *, n_cols, block):
    xt = jnp.transpose(logits2_ref[...], (1, 0))
    half = block // 2
    hrows = half // _LANES

    def part(xs, lab):
        m = jnp.max(xs, axis=0, keepdims=True)
        s = jnp.sum(jnp.exp(xs - m), axis=0, keepdims=True)
        conf = 1.0 / s
        col = jax.lax.broadcasted_iota(jnp.int32, xs.shape, 0)
        pred = jnp.min(jnp.where(xs == m, col, n_cols), axis=0, keepdims=True)
        conf2 = conf.reshape(hrows, _LANES)
        accb = pred.reshape(hrows, _LANES) == lab
        return jnp.where(accb, -conf2, conf2)

    packed_ref[0:hrows] = part(xt[0:n_cols], labe_ref[...])
    packed_ref[hrows:2 * hrows] = part(xt[n_cols:2 * n_cols], labo_ref[...])


def _sc_hist(bb_hbm, packed_hbm, ci_hbm, cf_hbm, bb_v, pk_v, hci, hcf, *, chunk, nc):
    wid = lax.axis_index("s") * nc + lax.axis_index("c")
    pltpu.sync_copy(bb_hbm, bb_v)
    pltpu.sync_copy(packed_hbm.at[pl.ds(wid * chunk, chunk)], pk_v)

    zi = jnp.zeros((16,), jnp.int32)
    zf = jnp.zeros((16,), jnp.float32)
    for k in range(16):
        hci[pl.ds(k * 16, 16)] = zi
        hcf[pl.ds(k * 16, 16)] = zf

    lanei = lax.broadcasted_iota(jnp.int32, (16,), 0)
    one = jnp.int32(1)
    zero = jnp.int32(0)

    def step(t, carry):
        p = pk_v[pl.ds(t * 16, 16)]
        ai = jnp.where(p < 0.0, one, zero)
        c = jnp.abs(p)
        t15 = c * jnp.float32(_N_BINS)
        fi = t15.astype(jnp.int32)
        notint = jnp.where(t15 != fi.astype(jnp.float32), one, zero)
        b0 = jnp.clip(fi + notint - 1, 0, _N_BINS - 1)
        lo = plsc.load_gather(bb_v, [b0])
        b0 = b0 - jnp.where(c <= lo, one, zero)
        up = plsc.load_gather(bb_v, [b0 + 1])
        b0 = b0 + jnp.where(c > up, one, zero)
        idx = b0 * 16 + lanei
        plsc.addupdate_scatter(hci, [idx], 1 + ai * 65536)
        plsc.addupdate_scatter(hcf, [idx], c)
        return carry

    lax.fori_loop(0, chunk // 16, step, 0)

    pltpu.sync_copy(hci, ci_hbm.at[wid])
    pltpu.sync_copy(hcf, cf_hbm.at[wid])


def _final_body(ci_ref, cf_ref, out_ref, *, n_total):
    ci = ci_ref[...]
    cnt_f = (ci & 0xFFFF).astype(jnp.float32)
    acc_f = (ci >> 16).astype(jnp.float32)
    conf_f = cf_ref[...]
    cnt_r = jnp.sum(cnt_f, axis=0, keepdims=True)
    acc_r = jnp.sum(acc_f, axis=0, keepdims=True)
    conf_r = jnp.sum(conf_f, axis=0, keepdims=True)

    def bins(row):
        return jnp.concatenate(
            [jnp.sum(row[:, b * 16:(b + 1) * 16], axis=1, keepdims=True)
             for b in range(16)], axis=0)

    cnt = bins(cnt_r)
    accs = bins(acc_r)
    confs = bins(conf_r)
    denom = jnp.maximum(cnt, 1.0)
    acc_in = accs / denom
    conf_in = confs / denom
    gap = jnp.abs(conf_in - acc_in)
    nonempty = cnt > 0.0
    ece = jnp.sum(jnp.where(nonempty, gap * (cnt / n_total), 0.0))
    mce = jnp.max(jnp.where(nonempty, gap, -jnp.inf))
    mce = jnp.where(jnp.isneginf(mce), jnp.float32(1.0), mce)
    lane = jax.lax.broadcasted_iota(jnp.int32, (1, _LANES), 1)
    out_ref[...] = jnp.where(lane == 0, ece, jnp.where(lane == 1, mce, 0.0))


def kernel(logits, labels):
    n, c = logits.shape
    block = min(_BLOCK, n)
    grid = n // block
    half = block // 2
    hrows = half // _LANES
    logits2 = logits.reshape(n // 2, 2 * c)
    lab2 = labels.astype(jnp.int32).reshape(n // 2, 2)
    labe = lab2[:, 0].reshape(n // 2 // _LANES, _LANES)
    labo = lab2[:, 1].reshape(n // 2 // _LANES, _LANES)

    dense = functools.partial(_dense_body, n_cols=c, block=block)
    packed = pl.pallas_call(
        dense,
        grid=(grid,),
        in_specs=[
            pl.BlockSpec((half, 2 * c), lambda i: (i, 0)),
            pl.BlockSpec((hrows, _LANES), lambda i: (i, 0)),
            pl.BlockSpec((hrows, _LANES), lambda i: (i, 0)),
        ],
        out_specs=pl.BlockSpec((2 * hrows, _LANES), lambda i: (i, 0)),
        out_shape=jax.ShapeDtypeStruct((n // _LANES, _LANES), jnp.float32),
    )(logits2, labe, labo)
    packed_flat = packed.reshape(n)

    info = plsc.get_sparse_core_info()
    nc_, ns_ = info.num_cores, info.num_subcores
    nw = nc_ * ns_
    chunk = n // nw
    bb = jnp.linspace(0.0, 1.0, _N_BINS + 1).astype(jnp.float32)
    mesh = plsc.VectorSubcoreMesh(core_axis_name="c", subcore_axis_name="s")
    sc = functools.partial(_sc_hist, chunk=chunk, nc=nc_)
    sc_call = functools.partial(
        pl.kernel, mesh=mesh,
        compiler_params=pltpu.CompilerParams(needs_layout_passes=False),
        out_type=[
            jax.ShapeDtypeStruct((nw, 256), jnp.int32),
            jax.ShapeDtypeStruct((nw, 256), jnp.float32),
        ],
        scratch_types=[
            pltpu.VMEM((_N_BINS + 1,), jnp.float32),
            pltpu.VMEM((chunk,), jnp.float32),
            pltpu.VMEM((256,), jnp.int32),
            pltpu.VMEM((256,), jnp.float32),
        ],
    )(sc)
    ci, cf = sc_call(bb, packed_flat)

    fin = functools.partial(_final_body, n_total=float(n))
    out = pl.pallas_call(
        fin,
        grid=(1,),
        in_specs=[
            pl.BlockSpec((nw, 256), lambda i: (0, 0)),
            pl.BlockSpec((nw, 256), lambda i: (0, 0)),
        ],
        out_specs=pl.BlockSpec((1, _LANES), lambda i: (0, 0)),
        out_shape=jax.ShapeDtypeStruct((1, _LANES), jnp.float32),
    )(ci, cf)

    ece = out[0, 0:1]
    mce = out[0, 1]
    return (ece, mce)


# speedup vs baseline: 1.9563x; 1.9563x over previous
import functools

import jax
import jax.numpy as jnp
from jax import lax
from jax.experimental import pallas as pl
from jax.experimental.pallas import tpu as pltpu
from jax.experimental.pallas import tpu_sc as plsc

_N_BINS = 15
_LANES = 128
_BLOCK = 16384


def _dense_body(logits_ref, labels_ref, packed_ref, *, n_cols, block):
    xt = jnp.transpose(logits_ref[...], (1, 0))
    m = jnp.max(xt, axis=0, keepdims=True)
    s = jnp.sum(jnp.exp(xt - m), axis=0, keepdims=True)
    conf = 1.0 / s
    col = jax.lax.broadcasted_iota(jnp.int32, xt.shape, 0)
    pred = jnp.min(jnp.where(xt == m, col, n_cols), axis=0, keepdims=True)
    rows = block // _LANES
    conf2 = conf.reshape(rows, _LANES)
    accb = pred.reshape(rows, _LANES) == labels_ref[...]
    packed_ref[...] = jnp.where(accb, -conf2, conf2)


def _sc_hist(bb_hbm, packed_hbm, ci_hbm, cf_hbm, bb_v, pk_v, hci, hcf, *, chunk, nc):
    wid = lax.axis_index("s") * nc + lax.axis_index("c")
    pltpu.sync_copy(bb_hbm, bb_v)
    pltpu.sync_copy(packed_hbm.at[pl.ds(wid * chunk, chunk)], pk_v)

    zi = jnp.zeros((16,), jnp.int32)
    zf = jnp.zeros((16,), jnp.float32)
    for k in range(16):
        hci[pl.ds(k * 16, 16)] = zi
        hcf[pl.ds(k * 16, 16)] = zf

    lanei = lax.broadcasted_iota(jnp.int32, (16,), 0)
    one = jnp.int32(1)
    zero = jnp.int32(0)

    def step(t, carry):
        p = pk_v[pl.ds(t * 16, 16)]
        ai = jnp.where(p < 0.0, one, zero)
        c = jnp.abs(p)
        t15 = c * jnp.float32(_N_BINS)
        fi = t15.astype(jnp.int32)
        notint = jnp.where(t15 != fi.astype(jnp.float32), one, zero)
        b0 = jnp.clip(fi + notint - 1, 0, _N_BINS - 1)
        lo = plsc.load_gather(bb_v, [b0])
        b0 = b0 - jnp.where(c <= lo, one, zero)
        up = plsc.load_gather(bb_v, [b0 + 1])
        b0 = b0 + jnp.where(c > up, one, zero)
        idx = b0 * 16 + lanei
        plsc.addupdate_scatter(hci, [idx], 1 + ai * 65536)
        plsc.addupdate_scatter(hcf, [idx], c)
        return carry

    lax.fori_loop(0, chunk // 16, step, 0, unroll=8)

    pltpu.sync_copy(hci, ci_hbm.at[wid])
    pltpu.sync_copy(hcf, cf_hbm.at[wid])


def _final_body(ci_ref, cf_ref, out_ref, *, n_total):
    ci = ci_ref[...]
    cnt_f = (ci & 0xFFFF).astype(jnp.float32)
    acc_f = (ci >> 16).astype(jnp.float32)
    conf_f = cf_ref[...]
    cnt_r = jnp.sum(cnt_f, axis=0, keepdims=True)
    acc_r = jnp.sum(acc_f, axis=0, keepdims=True)
    conf_r = jnp.sum(conf_f, axis=0, keepdims=True)

    def bins(row):
        return jnp.concatenate(
            [jnp.sum(row[:, b * 16:(b + 1) * 16], axis=1, keepdims=True)
             for b in range(16)], axis=0)

    cnt = bins(cnt_r)
    accs = bins(acc_r)
    confs = bins(conf_r)
    denom = jnp.maximum(cnt, 1.0)
    acc_in = accs / denom
    conf_in = confs / denom
    gap = jnp.abs(conf_in - acc_in)
    nonempty = cnt > 0.0
    ece = jnp.sum(jnp.where(nonempty, gap * (cnt / n_total), 0.0))
    mce = jnp.max(jnp.where(nonempty, gap, -jnp.inf))
    mce = jnp.where(jnp.isneginf(mce), jnp.float32(1.0), mce)
    lane = jax.lax.broadcasted_iota(jnp.int32, (1, _LANES), 1)
    out_ref[...] = jnp.where(lane == 0, ece, jnp.where(lane == 1, mce, 0.0))


def kernel(logits, labels):
    n, c = logits.shape
    block = min(_BLOCK, n)
    rows = block // _LANES
    labels_r = labels.astype(jnp.int32).reshape(n // _LANES, _LANES)

    info = plsc.get_sparse_core_info()
    nc_, ns_ = info.num_cores, info.num_subcores
    nw = nc_ * ns_
    bb = jnp.linspace(0.0, 1.0, _N_BINS + 1).astype(jnp.float32)
    mesh = plsc.VectorSubcoreMesh(core_axis_name="c", subcore_axis_name="s")

    phases = 2 if n % (2 * block) == 0 else 1
    np_ = n // phases
    chunk = np_ // nw
    grid = np_ // block

    dense = functools.partial(_dense_body, n_cols=c, block=block)
    sc = functools.partial(_sc_hist, chunk=chunk, nc=nc_)
    sc_call = functools.partial(
        pl.kernel, mesh=mesh,
        compiler_params=pltpu.CompilerParams(needs_layout_passes=False),
        out_type=[
            jax.ShapeDtypeStruct((nw, 256), jnp.int32),
            jax.ShapeDtypeStruct((nw, 256), jnp.float32),
        ],
        scratch_types=[
            pltpu.VMEM((_N_BINS + 1,), jnp.float32),
            pltpu.VMEM((chunk,), jnp.float32),
            pltpu.VMEM((256,), jnp.int32),
            pltpu.VMEM((256,), jnp.float32),
        ],
    )(sc)

    cis, cfs = [], []
    for p in range(phases):
        lo = p * np_
        packed = pl.pallas_call(
            dense,
            grid=(grid,),
            in_specs=[
                pl.BlockSpec((block, c), lambda i, lo=lo: (lo // block + i, 0)),
                pl.BlockSpec((rows, _LANES),
                             lambda i, lo=lo: (lo // block + i, 0)),
            ],
            out_specs=pl.BlockSpec((rows, _LANES), lambda i: (i, 0)),
            out_shape=jax.ShapeDtypeStruct((np_ // _LANES, _LANES),
                                           jnp.float32),
        )(logits, labels_r)
        ci, cf = sc_call(bb, packed.reshape(np_))
        cis.append(ci)
        cfs.append(cf)

    ci_all = jnp.concatenate(cis, axis=0) if phases > 1 else cis[0]
    cf_all = jnp.concatenate(cfs, axis=0) if phases > 1 else cfs[0]

    fin = functools.partial(_final_body, n_total=float(n))
    out = pl.pallas_call(
        fin,
        grid=(1,),
        in_specs=[
            pl.BlockSpec((phases * nw, 256), lambda i: (0, 0)),
            pl.BlockSpec((phases * nw, 256), lambda i: (0, 0)),
        ],
        out_specs=pl.BlockSpec((1, _LANES), lambda i: (0, 0)),
        out_shape=jax.ShapeDtypeStruct((1, _LANES), jnp.float32),
    )(ci_all, cf_all)

    ece = out[0, 0:1]
    mce = out[0, 1]
    return (ece, mce)


# speedup vs baseline: 2.0154x; 1.0302x over previous
import functools

import jax
import jax.numpy as jnp
from jax import lax
from jax.experimental import pallas as pl
from jax.experimental.pallas import tpu as pltpu
from jax.experimental.pallas import tpu_sc as plsc

_N_BINS = 15
_LANES = 128
_BLOCK = 16384


def _dense_body(logits_ref, labels_ref, packed_ref, *, n_cols, block):
    xt = jnp.transpose(logits_ref[...], (1, 0))
    m = jnp.max(xt, axis=0, keepdims=True)
    s = jnp.sum(jnp.exp(xt - m), axis=0, keepdims=True)
    conf = 1.0 / s
    col = jax.lax.broadcasted_iota(jnp.int32, xt.shape, 0)
    pred = jnp.min(jnp.where(xt == m, col, n_cols), axis=0, keepdims=True)
    rows = block // _LANES
    conf2 = conf.reshape(rows, _LANES)
    accb = pred.reshape(rows, _LANES) == labels_ref[...]
    packed_ref[...] = jnp.where(accb, -conf2, conf2)


def _sc_hist(bb_hbm, packed_hbm, ci_hbm, cf_hbm, bb_v, pk_v, hci, hcf, *, chunk, nc):
    wid = lax.axis_index("s") * nc + lax.axis_index("c")
    pltpu.sync_copy(bb_hbm, bb_v)
    pltpu.sync_copy(packed_hbm.at[pl.ds(wid * chunk, chunk)], pk_v)

    zi = jnp.zeros((16,), jnp.int32)
    zf = jnp.zeros((16,), jnp.float32)
    for k in range(16):
        hci[pl.ds(k * 16, 16)] = zi
        hcf[pl.ds(k * 16, 16)] = zf

    lanei = lax.broadcasted_iota(jnp.int32, (16,), 0)
    one = jnp.int32(1)
    zero = jnp.int32(0)

    def step(t, carry):
        p = pk_v[pl.ds(t * 16, 16)]
        ai = jnp.where(p < 0.0, one, zero)
        c = jnp.abs(p)
        t15 = c * jnp.float32(_N_BINS)
        fi = t15.astype(jnp.int32)
        notint = jnp.where(t15 != fi.astype(jnp.float32), one, zero)
        b0 = jnp.clip(fi + notint - 1, 0, _N_BINS - 1)
        lo = plsc.load_gather(bb_v, [b0])
        b0 = b0 - jnp.where(c <= lo, one, zero)
        up = plsc.load_gather(bb_v, [b0 + 1])
        b0 = b0 + jnp.where(c > up, one, zero)
        idx = b0 * 16 + lanei
        plsc.addupdate_scatter(hci, [idx], 1 + ai * 65536)
        plsc.addupdate_scatter(hcf, [idx], c)
        return carry

    lax.fori_loop(0, chunk // 16, step, 0, unroll=8)

    pltpu.sync_copy(hci, ci_hbm.at[wid])
    pltpu.sync_copy(hcf, cf_hbm.at[wid])


def _final_body(ci_ref, cf_ref, out_ref, *, n_total):
    ci = ci_ref[...]
    cnt_f = (ci & 0xFFFF).astype(jnp.float32)
    acc_f = (ci >> 16).astype(jnp.float32)
    conf_f = cf_ref[...]
    cnt_r = jnp.sum(cnt_f, axis=0, keepdims=True)
    acc_r = jnp.sum(acc_f, axis=0, keepdims=True)
    conf_r = jnp.sum(conf_f, axis=0, keepdims=True)

    def bins(row):
        return jnp.concatenate(
            [jnp.sum(row[:, b * 16:(b + 1) * 16], axis=1, keepdims=True)
             for b in range(16)], axis=0)

    cnt = bins(cnt_r)
    accs = bins(acc_r)
    confs = bins(conf_r)
    denom = jnp.maximum(cnt, 1.0)
    acc_in = accs / denom
    conf_in = confs / denom
    gap = jnp.abs(conf_in - acc_in)
    nonempty = cnt > 0.0
    ece = jnp.sum(jnp.where(nonempty, gap * (cnt / n_total), 0.0))
    mce = jnp.max(jnp.where(nonempty, gap, -jnp.inf))
    mce = jnp.where(jnp.isneginf(mce), jnp.float32(1.0), mce)
    lane = jax.lax.broadcasted_iota(jnp.int32, (1, _LANES), 1)
    out_ref[...] = jnp.where(lane == 0, ece, jnp.where(lane == 1, mce, 0.0))


def kernel(logits, labels):
    n, c = logits.shape
    block = min(_BLOCK, n)
    rows = block // _LANES
    labels_r = labels.astype(jnp.int32).reshape(n // _LANES, _LANES)

    info = plsc.get_sparse_core_info()
    nc_, ns_ = info.num_cores, info.num_subcores
    nw = nc_ * ns_
    bb = jnp.linspace(0.0, 1.0, _N_BINS + 1).astype(jnp.float32)
    mesh = plsc.VectorSubcoreMesh(core_axis_name="c", subcore_axis_name="s")

    phases = 1
    for cand_p in (4, 2):
        if n % (cand_p * block) == 0 and (n // cand_p) % (nw * 16) == 0:
            phases = cand_p
            break
    np_ = n // phases
    chunk = np_ // nw
    grid = np_ // block

    dense = functools.partial(_dense_body, n_cols=c, block=block)
    sc = functools.partial(_sc_hist, chunk=chunk, nc=nc_)
    sc_call = functools.partial(
        pl.kernel, mesh=mesh,
        compiler_params=pltpu.CompilerParams(needs_layout_passes=False),
        out_type=[
            jax.ShapeDtypeStruct((nw, 256), jnp.int32),
            jax.ShapeDtypeStruct((nw, 256), jnp.float32),
        ],
        scratch_types=[
            pltpu.VMEM((_N_BINS + 1,), jnp.float32),
            pltpu.VMEM((chunk,), jnp.float32),
            pltpu.VMEM((256,), jnp.int32),
            pltpu.VMEM((256,), jnp.float32),
        ],
    )(sc)

    cis, cfs = [], []
    for p in range(phases):
        lo = p * np_
        packed = pl.pallas_call(
            dense,
            grid=(grid,),
            in_specs=[
                pl.BlockSpec((block, c), lambda i, lo=lo: (lo // block + i, 0)),
                pl.BlockSpec((rows, _LANES),
                             lambda i, lo=lo: (lo // block + i, 0)),
            ],
            out_specs=pl.BlockSpec((rows, _LANES), lambda i: (i, 0)),
            out_shape=jax.ShapeDtypeStruct((np_ // _LANES, _LANES),
                                           jnp.float32),
        )(logits, labels_r)
        ci, cf = sc_call(bb, packed.reshape(np_))
        cis.append(ci)
        cfs.append(cf)

    ci_all = jnp.concatenate(cis, axis=0) if phases > 1 else cis[0]
    cf_all = jnp.concatenate(cfs, axis=0) if phases > 1 else cfs[0]

    fin = functools.partial(_final_body, n_total=float(n))
    out = pl.pallas_call(
        fin,
        grid=(1,),
        in_specs=[
            pl.BlockSpec((phases * nw, 256), lambda i: (0, 0)),
            pl.BlockSpec((phases * nw, 256), lambda i: (0, 0)),
        ],
        out_specs=pl.BlockSpec((1, _LANES), lambda i: (0, 0)),
        out_shape=jax.ShapeDtypeStruct((1, _LANES), jnp.float32),
    )(ci_all, cf_all)

    ece = out[0, 0:1]
    mce = out[0, 1]
    return (ece, mce)
